# baseline (speedup 1.0000x reference)
import functools

import jax
import jax.numpy as jnp
from jax import lax
from jax.experimental import pallas as pl
from jax.experimental.pallas import tpu as pltpu
from jax.experimental.pallas import tpu_sc as plsc

_NC = 2
_NS = 16
_NW = _NC * _NS
_L = 16

_FIELD_SIZE = 40000


@functools.partial(jax.jit, static_argnums=(3, 4))
def _sc_lookup_sum(x_prep, table_flat, bias_b, B, F):
    rpt = B // _NW
    chunk = F * rpt
    n_slices = rpt // _L

    mesh = plsc.VectorSubcoreMesh(
        core_axis_name="c", subcore_axis_name="s",
        num_cores=_NC, num_subcores=_NS)

    @functools.partial(
        pl.kernel,
        out_type=jax.ShapeDtypeStruct((B,), jnp.float32),
        mesh=mesh,
        scratch_types=[
            pltpu.VMEM((chunk,), jnp.int32),
            pltpu.VMEM((chunk,), jnp.float32),
            pltpu.VMEM((rpt,), jnp.float32),
            pltpu.VMEM((_L,), jnp.float32),
            pltpu.SemaphoreType.DMA,
        ],
    )
    def body(x_hbm, table_hbm, bias_hbm, out_hbm, idx_v, rows_v,
             out_v, bias_v, sem):
        wid = lax.axis_index("s") * _NC + lax.axis_index("c")
        base = wid * chunk

        pltpu.sync_copy(bias_hbm, bias_v)
        pltpu.sync_copy(x_hbm.at[pl.ds(base, chunk)], idx_v)

        def add_off(i, _):
            off = (i // n_slices) * _FIELD_SIZE
            j = pl.multiple_of(i * _L, _L)
            idx_v[pl.ds(j, _L)] = idx_v[pl.ds(j, _L)] + off
            return 0

        lax.fori_loop(0, F * n_slices, add_off, 0)

        pltpu.async_copy(table_hbm.at[idx_v], rows_v, sem).wait()

        def reduce_rows(i, _):
            j = pl.multiple_of(i * _L, _L)
            acc = bias_v[...]
            for f in range(F):
                acc = acc + rows_v[pl.ds(f * rpt + j, _L)]
            out_v[pl.ds(j, _L)] = acc
            return 0

        lax.fori_loop(0, n_slices, reduce_rows, 0)

        pltpu.sync_copy(out_v, out_hbm.at[pl.ds(wid * rpt, rpt)])

    return body(x_prep, table_flat, bias_b)


def kernel(x, table, bias):
    B, F = x.shape
    x_prep = (x.astype(jnp.int32)
              .reshape(_NW, B // _NW, F)
              .transpose(0, 2, 1)
              .reshape(-1))
    table_flat = table.reshape(-1)
    bias_b = jnp.broadcast_to(bias.astype(jnp.float32), (_L,))
    out = _sc_lookup_sum(x_prep, table_flat, bias_b, B, F)
    return out.reshape(B, 1)


# speedup vs baseline: 1.0378x; 1.0378x over previous
import functools

import jax
import jax.numpy as jnp
from jax import lax
from jax.experimental import pallas as pl
from jax.experimental.pallas import tpu as pltpu
from jax.experimental.pallas import tpu_sc as plsc

_NC = 2
_NS = 16
_NW = _NC * _NS
_L = 16

_FIELD_SIZE = 40000


@functools.partial(jax.jit, static_argnums=(3, 4))
def _sc_lookup_sum(x_prep, table_flat, bias_b, B, F):
    rpt = B // _NW
    chunk = F * rpt
    n_slices = rpt // _L
    tbl_n = F * _FIELD_SIZE
    share = tbl_n // _NS

    mesh = plsc.VectorSubcoreMesh(
        core_axis_name="c", subcore_axis_name="s",
        num_cores=_NC, num_subcores=_NS)

    @functools.partial(
        pl.kernel,
        out_type=jax.ShapeDtypeStruct((B,), jnp.float32),
        mesh=mesh,
        scratch_types=[
            pltpu.VMEM_SHARED((tbl_n,), jnp.float32),
            pltpu.VMEM((tbl_n // (_NS * 5),), jnp.float32),
            pltpu.VMEM((chunk,), jnp.int32),
            pltpu.VMEM((chunk,), jnp.float32),
            pltpu.VMEM((rpt,), jnp.float32),
            pltpu.VMEM((_L,), jnp.float32),
            pltpu.SemaphoreType.DMA,
            pltpu.SemaphoreType.DMA,
        ],
    )
    def body(x_hbm, table_hbm, bias_hbm, out_hbm, tbl_s, stage_v, idx_v,
             rows_v, out_v, bias_v, sem, tsem):
        wid = lax.axis_index("s") * _NC + lax.axis_index("c")
        sid = lax.axis_index("s")
        base = wid * chunk

        piece = share // 5

        def stage(k, _):
            src = pl.ds(sid * share + k * piece, piece)
            pltpu.sync_copy(table_hbm.at[src], stage_v)
            pltpu.sync_copy(stage_v, tbl_s.at[src])
            return 0

        lax.fori_loop(0, 5, stage, 0)

        pltpu.sync_copy(bias_hbm, bias_v)
        pltpu.sync_copy(x_hbm.at[pl.ds(base, chunk)], idx_v)

        def add_off(i, _):
            off = (i // n_slices) * _FIELD_SIZE
            j = pl.multiple_of(i * _L, _L)
            idx_v[pl.ds(j, _L)] = idx_v[pl.ds(j, _L)] + off
            return 0

        lax.fori_loop(0, F * n_slices, add_off, 0)

        plsc.subcore_barrier()

        pltpu.async_copy(tbl_s.at[idx_v], rows_v, sem).wait()

        def reduce_rows(i, _):
            j = pl.multiple_of(i * _L, _L)
            acc = bias_v[...]
            for f in range(F):
                acc = acc + rows_v[pl.ds(f * rpt + j, _L)]
            out_v[pl.ds(j, _L)] = acc
            return 0

        lax.fori_loop(0, n_slices, reduce_rows, 0)

        pltpu.sync_copy(out_v, out_hbm.at[pl.ds(wid * rpt, rpt)])

    return body(x_prep, table_flat, bias_b)


def kernel(x, table, bias):
    B, F = x.shape
    x_prep = (x.astype(jnp.int32)
              .reshape(_NW, B // _NW, F)
              .transpose(0, 2, 1)
              .reshape(-1))
    table_flat = table.reshape(-1)
    bias_b = jnp.broadcast_to(bias.astype(jnp.float32), (_L,))
    out = _sc_lookup_sum(x_prep, table_flat, bias_b, B, F)
    return out.reshape(B, 1)


# speedup vs baseline: 1.1113x; 1.0708x over previous
import functools

import jax
import jax.numpy as jnp
from jax import lax
from jax.experimental import pallas as pl
from jax.experimental.pallas import tpu as pltpu
from jax.experimental.pallas import tpu_sc as plsc

_NC = 2
_NS = 16
_NW = _NC * _NS
_L = 16

_FIELD_SIZE = 40000
_PIECES = 5


@functools.partial(jax.jit, static_argnums=(3, 4))
def _sc_lookup_sum(x_prep, table_flat, bias_b, B, F):
    rpt = B // _NW
    chunk = F * rpt
    half = chunk // 2
    fh = F // 2
    n_slices = rpt // _L
    tbl_n = F * _FIELD_SIZE
    share = tbl_n // _NS
    piece = share // _PIECES

    mesh = plsc.VectorSubcoreMesh(
        core_axis_name="c", subcore_axis_name="s",
        num_cores=_NC, num_subcores=_NS)

    @functools.partial(
        pl.kernel,
        out_type=jax.ShapeDtypeStruct((B,), jnp.float32),
        mesh=mesh,
        scratch_types=[
            pltpu.VMEM_SHARED((tbl_n,), jnp.float32),
            pltpu.VMEM((piece,), jnp.float32),
            pltpu.VMEM((piece,), jnp.float32),
            pltpu.VMEM((half,), jnp.int32),
            pltpu.VMEM((half,), jnp.int32),
            pltpu.VMEM((half,), jnp.float32),
            pltpu.VMEM((half,), jnp.float32),
            pltpu.VMEM((rpt,), jnp.float32),
            pltpu.VMEM((_L,), jnp.float32),
            pltpu.SemaphoreType.DMA,
            pltpu.SemaphoreType.DMA,
            pltpu.SemaphoreType.DMA,
            pltpu.SemaphoreType.DMA,
            pltpu.SemaphoreType.DMA,
            pltpu.SemaphoreType.DMA,
        ],
    )
    def body(x_hbm, table_hbm, bias_hbm, out_hbm, tbl_s, stage_a, stage_b,
             idx_a, idx_b, rows_a, rows_b, out_v, bias_v,
             sem_i, sem_ha, sem_hb, sem_sa, sem_sb, sem_g):
        wid = lax.axis_index("s") * _NC + lax.axis_index("c")
        sid = lax.axis_index("s")
        base = wid * chunk

        ia = pltpu.async_copy(x_hbm.at[pl.ds(base, half)], idx_a, sem_i)
        ib = pltpu.async_copy(x_hbm.at[pl.ds(base + half, half)], idx_b,
                              sem_i)
        pltpu.sync_copy(bias_hbm, bias_v)

        def tsrc(k):
            return table_hbm.at[pl.ds(sid * share + k * piece, piece)]

        def tdst(k):
            return tbl_s.at[pl.ds(sid * share + k * piece, piece)]

        h0 = pltpu.async_copy(tsrc(0), stage_a, sem_ha)
        h1 = pltpu.async_copy(tsrc(1), stage_b, sem_hb)
        h0.wait()
        s0 = pltpu.async_copy(stage_a, tdst(0), sem_sa)
        h1.wait()
        s1 = pltpu.async_copy(stage_b, tdst(1), sem_sb)
        s0.wait()
        h2 = pltpu.async_copy(tsrc(2), stage_a, sem_ha)
        s1.wait()
        h3 = pltpu.async_copy(tsrc(3), stage_b, sem_hb)
        h2.wait()
        s2 = pltpu.async_copy(stage_a, tdst(2), sem_sa)
        h3.wait()
        s3 = pltpu.async_copy(stage_b, tdst(3), sem_sb)
        s2.wait()
        h4 = pltpu.async_copy(tsrc(4), stage_a, sem_ha)
        h4.wait()
        s4 = pltpu.async_copy(stage_a, tdst(4), sem_sa)

        ia.wait()
        ib.wait()

        def add_off(i, _):
            j = pl.multiple_of(i * _L, _L)
            for f in range(1, fh):
                k = pl.ds(f * rpt + j, _L)
                idx_a[k] = idx_a[k] + f * _FIELD_SIZE
            for f in range(fh, F):
                k = pl.ds((f - fh) * rpt + j, _L)
                idx_b[k] = idx_b[k] + f * _FIELD_SIZE
            return 0

        lax.fori_loop(0, n_slices, add_off, 0)

        s3.wait()
        s4.wait()
        plsc.subcore_barrier()

        g1 = pltpu.async_copy(tbl_s.at[idx_a], rows_a, sem_g)
        g2 = pltpu.async_copy(tbl_s.at[idx_b], rows_b, sem_g)
        g1.wait()
        g2.wait()

        def reduce_rows(i, _):
            j = pl.multiple_of(i * _L, _L)
            acc = bias_v[...]
            for f in range(fh):
                acc = acc + rows_a[pl.ds(f * rpt + j, _L)]
            for f in range(fh):
                acc = acc + rows_b[pl.ds(f * rpt + j, _L)]
            out_v[pl.ds(j, _L)] = acc
            return 0

        lax.fori_loop(0, n_slices, reduce_rows, 0)

        pltpu.sync_copy(out_v, out_hbm.at[pl.ds(wid * rpt, rpt)])

    return body(x_prep, table_flat, bias_b)


def kernel(x, table, bias):
    B, F = x.shape
    x_prep = (x.astype(jnp.int32)
              .reshape(_NW, B // _NW, F)
              .transpose(0, 2, 1)
              .reshape(-1))
    table_flat = table.reshape(-1)
    bias_b = jnp.broadcast_to(bias.astype(jnp.float32), (_L,))
    out = _sc_lookup_sum(x_prep, table_flat, bias_b, B, F)
    return out.reshape(B, 1)


# speedup vs baseline: 1.1115x; 1.0001x over previous
import functools

import jax
import jax.numpy as jnp
from jax import lax
from jax.experimental import pallas as pl
from jax.experimental.pallas import tpu as pltpu
from jax.experimental.pallas import tpu_sc as plsc

_NC = 2
_NS = 16
_NW = _NC * _NS
_L = 16

_FIELD_SIZE = 40000
_PIECES = 5


@functools.partial(jax.jit, static_argnums=(3, 4))
def _sc_lookup_sum(x_prep, table_flat, bias_b, B, F):
    rpt = B // _NW
    chunk = F * rpt
    fcuts = (0, 7, 13, 20, 26)
    fcnt = tuple(fcuts[i + 1] - fcuts[i] for i in range(4))
    glen = tuple(c * rpt for c in fcnt)
    goff = tuple(fcuts[i] * rpt for i in range(4))
    n_slices = rpt // _L
    tbl_n = F * _FIELD_SIZE
    share = tbl_n // _NS
    piece = share // _PIECES

    mesh = plsc.VectorSubcoreMesh(
        core_axis_name="c", subcore_axis_name="s",
        num_cores=_NC, num_subcores=_NS)

    @functools.partial(
        pl.kernel,
        out_type=jax.ShapeDtypeStruct((B,), jnp.float32),
        mesh=mesh,
        scratch_types=[
            pltpu.VMEM_SHARED((tbl_n,), jnp.float32),
            pltpu.VMEM((piece,), jnp.float32),
            pltpu.VMEM((piece,), jnp.float32),
            pltpu.VMEM((glen[0],), jnp.int32),
            pltpu.VMEM((glen[1],), jnp.int32),
            pltpu.VMEM((glen[2],), jnp.int32),
            pltpu.VMEM((glen[3],), jnp.int32),
            pltpu.VMEM((glen[0],), jnp.float32),
            pltpu.VMEM((glen[1],), jnp.float32),
            pltpu.VMEM((glen[2],), jnp.float32),
            pltpu.VMEM((glen[3],), jnp.float32),
            pltpu.VMEM((rpt,), jnp.float32),
            pltpu.VMEM((_L,), jnp.float32),
            pltpu.SemaphoreType.DMA,
            pltpu.SemaphoreType.DMA,
            pltpu.SemaphoreType.DMA,
            pltpu.SemaphoreType.DMA,
            pltpu.SemaphoreType.DMA,
            pltpu.SemaphoreType.DMA,
        ],
    )
    def body(x_hbm, table_hbm, bias_hbm, out_hbm, tbl_s, stage_a, stage_b,
             idx_0, idx_1, idx_2, idx_3, rows_0, rows_1, rows_2, rows_3,
             out_v, bias_v, sem_i, sem_ha, sem_hb, sem_sa, sem_sb, sem_g):
        idx = (idx_0, idx_1, idx_2, idx_3)
        rows = (rows_0, rows_1, rows_2, rows_3)
        wid = lax.axis_index("s") * _NC + lax.axis_index("c")
        sid = lax.axis_index("s")
        base = wid * chunk

        icopies = [
            pltpu.async_copy(x_hbm.at[pl.ds(base + goff[g], glen[g])],
                             idx[g], sem_i)
            for g in range(4)]
        pltpu.sync_copy(bias_hbm, bias_v)

        def tsrc(k):
            return table_hbm.at[pl.ds(sid * share + k * piece, piece)]

        def tdst(k):
            return tbl_s.at[pl.ds(sid * share + k * piece, piece)]

        h0 = pltpu.async_copy(tsrc(0), stage_a, sem_ha)
        h1 = pltpu.async_copy(tsrc(1), stage_b, sem_hb)
        h0.wait()
        s0 = pltpu.async_copy(stage_a, tdst(0), sem_sa)
        h1.wait()
        s1 = pltpu.async_copy(stage_b, tdst(1), sem_sb)
        s0.wait()
        h2 = pltpu.async_copy(tsrc(2), stage_a, sem_ha)
        s1.wait()
        h3 = pltpu.async_copy(tsrc(3), stage_b, sem_hb)
        h2.wait()
        s2 = pltpu.async_copy(stage_a, tdst(2), sem_sa)
        h3.wait()
        s3 = pltpu.async_copy(stage_b, tdst(3), sem_sb)
        s2.wait()
        h4 = pltpu.async_copy(tsrc(4), stage_a, sem_ha)
        h4.wait()
        s4 = pltpu.async_copy(stage_a, tdst(4), sem_sa)

        for c in icopies:
            c.wait()

        def add_off(i, _):
            j = pl.multiple_of(i * _L, _L)
            for g in range(4):
                for f in range(fcuts[g], fcuts[g + 1]):
                    if f == 0:
                        continue
                    k = pl.ds((f - fcuts[g]) * rpt + j, _L)
                    idx[g][k] = idx[g][k] + f * _FIELD_SIZE
            return 0

        lax.fori_loop(0, n_slices, add_off, 0)

        s3.wait()
        s4.wait()
        plsc.subcore_barrier()

        gcopies = [pltpu.async_copy(tbl_s.at[idx[g]], rows[g], sem_g)
                   for g in range(4)]
        for c in gcopies:
            c.wait()

        def reduce_rows(i, _):
            j = pl.multiple_of(i * _L, _L)
            acc = bias_v[...]
            for g in range(4):
                for f in range(fcnt[g]):
                    acc = acc + rows[g][pl.ds(f * rpt + j, _L)]
            out_v[pl.ds(j, _L)] = acc
            return 0

        lax.fori_loop(0, n_slices, reduce_rows, 0)

        pltpu.sync_copy(out_v, out_hbm.at[pl.ds(wid * rpt, rpt)])

    return body(x_prep, table_flat, bias_b)


def kernel(x, table, bias):
    B, F = x.shape
    x_prep = (x.astype(jnp.int32)
              .reshape(_NW, B // _NW, F)
              .transpose(0, 2, 1)
              .reshape(-1))
    table_flat = table.reshape(-1)
    bias_b = jnp.broadcast_to(bias.astype(jnp.float32), (_L,))
    out = _sc_lookup_sum(x_prep, table_flat, bias_b, B, F)
    return out.reshape(B, 1)
